# Initial kernel scaffold; baseline (speedup 1.0000x reference)
import functools

import jax
import jax.numpy as jnp
from jax import lax
from jax.experimental import pallas as pl
from jax.experimental.pallas import tpu as pltpu
from jax.experimental.pallas import tpu_sc as plsc

_N = 10000
_E = 320000
_R = 4
_NUM_NEIGHBORS = 32

_NC = 2
_NS = 16
_NW = _NC * _NS

_G = 128
_EW = 10240
_E_PAD = _NW * _EW
_GW = _EW // _G
_N_ACC = 10240
_SLAB = _N_ACC // _NS


def _sc_aggregate(d: int):
    mesh = plsc.VectorSubcoreMesh(core_axis_name="c", subcore_axis_name="s")

    @functools.partial(
        pl.kernel,
        out_type=jax.ShapeDtypeStruct((_NC, _N_ACC, d), jnp.float32),
        mesh=mesh,
        scratch_types=[
            pltpu.VMEM((_GW, _G), jnp.int32),
            pltpu.VMEM((_GW, _G), jnp.int32),
            pltpu.VMEM((_GW, _G), jnp.int32),
            pltpu.VMEM((_GW, _G), jnp.int32),
            pltpu.VMEM((_G, d), jnp.float32),
            pltpu.VMEM_SHARED((_N_ACC, d), jnp.float32),
            pltpu.SemaphoreType.DMA,
        ],
    )
    def k(table_hbm, src_hbm, et_hbm, dst_hbm, out_hbm,
          src_v, et_v, dst_v, gidx_v, rows_v, acc, sem):
        c = lax.axis_index("c")
        s = lax.axis_index("s")
        wid = s * _NC + c
        base_g = wid * _GW

        pltpu.sync_copy(src_hbm.at[pl.ds(base_g, _GW)], src_v)
        pltpu.sync_copy(et_hbm.at[pl.ds(base_g, _GW)], et_v)
        pltpu.sync_copy(dst_hbm.at[pl.ds(base_g, _GW)], dst_v)

        def zrow(i, carry):
            for j in range(d // 16):
                rows_v[i, pl.ds(j * 16, 16)] = jnp.zeros((16,), jnp.float32)
            return carry
        lax.fori_loop(0, _G, zrow, 0)

        def zslab(i, carry):
            pltpu.sync_copy(rows_v, acc.at[pl.ds(s * _SLAB + i * _G, _G)])
            return carry
        lax.fori_loop(0, _SLAB // _G, zslab, 0)

        def gidx(g, carry):
            for j in range(_G // 16):
                sv = src_v[g, pl.ds(j * 16, 16)]
                tv = et_v[g, pl.ds(j * 16, 16)]
                gidx_v[g, pl.ds(j * 16, 16)] = sv * _R + tv
            return carry
        lax.fori_loop(0, _GW, gidx, 0)

        plsc.subcore_barrier()

        def body(g, carry):
            pltpu.async_copy(table_hbm.at[gidx_v.at[g]], rows_v, sem).wait()
            pltpu.sync_copy(rows_v, acc.at[dst_v.at[g]], add=True)
            return carry
        lax.fori_loop(0, _GW, body, 0)

        plsc.subcore_barrier()

        def cp(i, carry):
            off = s * _SLAB + i * _G
            pltpu.sync_copy(acc.at[pl.ds(off, _G)], out_hbm.at[c, pl.ds(off, _G)])
            return carry
        lax.fori_loop(0, _SLAB // _G, cp, 0)

    return k


def _mm1_kernel(x_ref, w_ref, o_ref):
    o_ref[...] = jnp.dot(x_ref[...], w_ref[...],
                         preferred_element_type=jnp.float32)


def _layer2_kernel(inv_n, p_ref, b_ref, w_ref, o_ref):
    h = jnp.maximum((p_ref[0] + p_ref[1]) * inv_n + b_ref[...], 0.0)
    o_ref[...] = jnp.dot(h, w_ref[...], preferred_element_type=jnp.float32)


def _final_kernel(inv_n, q_ref, b_ref, o_ref):
    o = (q_ref[0] + q_ref[1]) * inv_n + b_ref[...]
    m = jnp.max(o, axis=1, keepdims=True)
    lse = jnp.log(jnp.sum(jnp.exp(o - m), axis=1, keepdims=True)) + m
    o_ref[...] = o - lse


def kernel(x, edge_index, edge_type, W1, b1, W2, b2):
    nfeat = x.shape[1]
    nhid = W1.shape[2]
    nclass = W2.shape[2]
    inv_n = 1.0 / float(_NUM_NEIGHBORS)

    pad = _E_PAD - _E
    src = jnp.concatenate([edge_index[0].astype(jnp.int32),
                           jnp.zeros((pad,), jnp.int32)]).reshape(_NW * _GW, _G)
    dst = jnp.concatenate([edge_index[1].astype(jnp.int32),
                           jnp.full((pad,), _N, jnp.int32)]).reshape(_NW * _GW, _G)
    et = jnp.concatenate([edge_type.astype(jnp.int32),
                          jnp.zeros((pad,), jnp.int32)]).reshape(_NW * _GW, _G)
    W1f = W1.transpose(1, 0, 2).reshape(nfeat, _R * nhid)
    W2f = W2.transpose(1, 0, 2).reshape(nhid, _R * nclass)

    t1 = pl.pallas_call(
        _mm1_kernel,
        grid=(10,),
        in_specs=[pl.BlockSpec((_N // 10, nfeat), lambda i: (i, 0)),
                  pl.BlockSpec((nfeat, _R * nhid), lambda i: (0, 0))],
        out_specs=pl.BlockSpec((_N // 10, _R * nhid), lambda i: (i, 0)),
        out_shape=jax.ShapeDtypeStruct((_N, _R * nhid), jnp.float32),
    )(x, W1f)
    table1 = t1.reshape(_N * _R, nhid)

    p = _sc_aggregate(nhid)(table1, src, et, dst)

    t2 = pl.pallas_call(
        functools.partial(_layer2_kernel, inv_n),
        grid=(10,),
        in_specs=[pl.BlockSpec((_NC, _N_ACC // 10, nhid), lambda i: (0, i, 0)),
                  pl.BlockSpec((1, nhid), lambda i: (0, 0)),
                  pl.BlockSpec((nhid, _R * nclass), lambda i: (0, 0))],
        out_specs=pl.BlockSpec((_N_ACC // 10, _R * nclass), lambda i: (i, 0)),
        out_shape=jax.ShapeDtypeStruct((_N_ACC, _R * nclass), jnp.float32),
    )(p, b1.reshape(1, nhid), W2f)
    table2 = t2.reshape(_N_ACC * _R, nclass)

    q = _sc_aggregate(nclass)(table2, src, et, dst)

    out = pl.pallas_call(
        functools.partial(_final_kernel, inv_n),
        grid=(10,),
        in_specs=[pl.BlockSpec((_NC, _N_ACC // 10, nclass), lambda i: (0, i, 0)),
                  pl.BlockSpec((1, nclass), lambda i: (0, 0))],
        out_specs=pl.BlockSpec((_N_ACC // 10, nclass), lambda i: (i, 0)),
        out_shape=jax.ShapeDtypeStruct((_N_ACC, nclass), jnp.float32),
    )(q, b2.reshape(1, nclass))
    return out[:_N]


# speedup vs baseline: 10.5266x; 10.5266x over previous
import functools

import jax
import jax.numpy as jnp
from jax import lax
from jax.experimental import pallas as pl
from jax.experimental.pallas import tpu as pltpu
from jax.experimental.pallas import tpu_sc as plsc

_N = 10000
_E = 320000
_R = 4
_NUM_NEIGHBORS = 32

_NC = 2
_NS = 16
_NW = _NC * _NS

_G = 128
_EW = 10240
_E_PAD = _NW * _EW
_GW = _EW // _G
_N_ACC = 10240
_SLAB = _N_ACC // _NS


def _sc_aggregate(d: int):
    mesh = plsc.VectorSubcoreMesh(core_axis_name="c", subcore_axis_name="s")

    @functools.partial(
        pl.kernel,
        out_type=jax.ShapeDtypeStruct((_NC, _N_ACC, d), jnp.float32),
        mesh=mesh,
        scratch_types=[
            pltpu.VMEM((_GW, _G), jnp.int32),
            pltpu.VMEM((16, _G), jnp.int32),
            pltpu.VMEM((_GW, _G), jnp.int32),
            pltpu.VMEM((_G, d), jnp.float32),
            pltpu.VMEM_SHARED((_N_ACC, d), jnp.float32),
            pltpu.SemaphoreType.DMA,
        ],
    )
    def k(table_hbm, src_hbm, et_hbm, dst_hbm, out_hbm,
          gidx_v, et_stage, dst_v, rows_v, acc, sem):
        c = lax.axis_index("c")
        s = lax.axis_index("s")
        wid = s * _NC + c
        base_g = wid * _GW

        pltpu.sync_copy(src_hbm.at[pl.ds(base_g, _GW)], gidx_v)
        pltpu.sync_copy(dst_hbm.at[pl.ds(base_g, _GW)], dst_v)

        def zrow(i, carry):
            for j in range(d // 16):
                rows_v[i, pl.ds(j * 16, 16)] = jnp.zeros((16,), jnp.float32)
            return carry
        lax.fori_loop(0, _G, zrow, 0)

        def zslab(i, carry):
            pltpu.sync_copy(rows_v, acc.at[pl.ds(s * _SLAB + i * _G, _G)])
            return carry
        lax.fori_loop(0, _SLAB // _G, zslab, 0)

        def gchunk(ci, carry):
            pltpu.sync_copy(et_hbm.at[pl.ds(base_g + ci * 16, 16)], et_stage)

            def gidx(g, carry2):
                for j in range(_G // 16):
                    sv = gidx_v[ci * 16 + g, pl.ds(j * 16, 16)]
                    tv = et_stage[g, pl.ds(j * 16, 16)]
                    gidx_v[ci * 16 + g, pl.ds(j * 16, 16)] = sv * _R + tv
                return carry2
            lax.fori_loop(0, 16, gidx, 0)
            return carry
        lax.fori_loop(0, _GW // 16, gchunk, 0)

        plsc.subcore_barrier()

        def body(g, carry):
            pltpu.async_copy(table_hbm.at[gidx_v.at[g]], rows_v, sem).wait()
            pltpu.sync_copy(rows_v, acc.at[dst_v.at[g]], add=True)
            return carry
        lax.fori_loop(0, _GW, body, 0)

        plsc.subcore_barrier()

        def cp(i, carry):
            off = s * _SLAB + i * _G
            pltpu.sync_copy(acc.at[pl.ds(off, _G)], out_hbm.at[c, pl.ds(off, _G)])
            return carry
        lax.fori_loop(0, _SLAB // _G, cp, 0)

    return k


def _mm1_kernel(x_ref, w_ref, o_ref):
    o_ref[...] = jnp.dot(x_ref[...], w_ref[...],
                         preferred_element_type=jnp.float32)


def _layer2_kernel(inv_n, nclass, p_ref, b_ref, w_ref, o_ref):
    h = jnp.maximum((p_ref[0] + p_ref[1]) * inv_n + b_ref[...], 0.0)
    t = jnp.dot(h, w_ref[...], preferred_element_type=jnp.float32)
    z = jnp.zeros((h.shape[0], 128 - nclass), jnp.float32)
    for r in range(_R):
        o_ref[:, r * 128:r * 128 + nclass] = t[:, r * nclass:(r + 1) * nclass]
        o_ref[:, r * 128 + nclass:(r + 1) * 128] = z


def _final_kernel(inv_n, nclass, q_ref, b_ref, o_ref):
    o = (q_ref[0, :, :nclass] + q_ref[1, :, :nclass]) * inv_n + b_ref[...]
    m = jnp.max(o, axis=1, keepdims=True)
    lse = jnp.log(jnp.sum(jnp.exp(o - m), axis=1, keepdims=True)) + m
    o_ref[...] = o - lse


def kernel(x, edge_index, edge_type, W1, b1, W2, b2):
    nfeat = x.shape[1]
    nhid = W1.shape[2]
    nclass = W2.shape[2]
    inv_n = 1.0 / float(_NUM_NEIGHBORS)

    pad = _E_PAD - _E
    src = jnp.concatenate([edge_index[0].astype(jnp.int32),
                           jnp.zeros((pad,), jnp.int32)]).reshape(_NW * _GW, _G)
    dst = jnp.concatenate([edge_index[1].astype(jnp.int32),
                           jnp.full((pad,), _N, jnp.int32)]).reshape(_NW * _GW, _G)
    et = jnp.concatenate([edge_type.astype(jnp.int32),
                          jnp.zeros((pad,), jnp.int32)]).reshape(_NW * _GW, _G)
    W1f = W1.transpose(1, 0, 2).reshape(nfeat, _R * nhid)
    W2f = W2.transpose(1, 0, 2).reshape(nhid, _R * nclass)

    t1 = pl.pallas_call(
        _mm1_kernel,
        grid=(10,),
        in_specs=[pl.BlockSpec((_N // 10, nfeat), lambda i: (i, 0)),
                  pl.BlockSpec((nfeat, _R * nhid), lambda i: (0, 0))],
        out_specs=pl.BlockSpec((_N // 10, _R * nhid), lambda i: (i, 0)),
        out_shape=jax.ShapeDtypeStruct((_N, _R * nhid), jnp.float32),
    )(x, W1f)
    table1 = t1.reshape(_N * _R, nhid)

    p = _sc_aggregate(nhid)(table1, src, et, dst)

    t2 = pl.pallas_call(
        functools.partial(_layer2_kernel, inv_n, nclass),
        grid=(10,),
        in_specs=[pl.BlockSpec((_NC, _N_ACC // 10, nhid), lambda i: (0, i, 0)),
                  pl.BlockSpec((1, nhid), lambda i: (0, 0)),
                  pl.BlockSpec((nhid, _R * nclass), lambda i: (0, 0))],
        out_specs=pl.BlockSpec((_N_ACC // 10, _R * 128), lambda i: (i, 0)),
        out_shape=jax.ShapeDtypeStruct((_N_ACC, _R * 128), jnp.float32),
    )(p, b1.reshape(1, nhid), W2f)
    table2 = t2.reshape(_N_ACC * _R, 128)

    q = _sc_aggregate(128)(table2, src, et, dst)

    out = pl.pallas_call(
        functools.partial(_final_kernel, inv_n, nclass),
        grid=(10,),
        in_specs=[pl.BlockSpec((_NC, _N_ACC // 10, 128), lambda i: (0, i, 0)),
                  pl.BlockSpec((1, nclass), lambda i: (0, 0))],
        out_specs=pl.BlockSpec((_N_ACC // 10, nclass), lambda i: (i, 0)),
        out_shape=jax.ShapeDtypeStruct((_N_ACC, nclass), jnp.float32),
    )(q, b2.reshape(1, nclass))
    return out[:_N]


# speedup vs baseline: 11.3142x; 1.0748x over previous
import functools

import jax
import jax.numpy as jnp
from jax import lax
from jax.experimental import pallas as pl
from jax.experimental.pallas import tpu as pltpu
from jax.experimental.pallas import tpu_sc as plsc

_N = 10000
_E = 320000
_R = 4
_NUM_NEIGHBORS = 32

_NC = 2
_NS = 16
_NW = _NC * _NS

_G = 128
_EW = 10240
_E_PAD = _NW * _EW
_GW = _EW // _G
_HGW = _GW // 2
_N_ACC = 10240
_SLAB = _N_ACC // _NS


def _sc_aggregate(d: int):
    mesh = plsc.VectorSubcoreMesh(core_axis_name="c", subcore_axis_name="s")

    @functools.partial(
        pl.kernel,
        out_type=jax.ShapeDtypeStruct((_NC, _N_ACC, d), jnp.float32),
        mesh=mesh,
        scratch_types=[
            pltpu.VMEM((_HGW, _G), jnp.int32),
            pltpu.VMEM((_HGW, _G), jnp.int32),
            pltpu.VMEM((8, _G), jnp.int32),
            pltpu.VMEM((_G, d), jnp.float32),
            pltpu.VMEM((_G, d), jnp.float32),
            pltpu.VMEM_SHARED((_N_ACC, d), jnp.float32),
            pltpu.SemaphoreType.DMA,
            pltpu.SemaphoreType.DMA,
            pltpu.SemaphoreType.DMA,
            pltpu.SemaphoreType.DMA,
            pltpu.SemaphoreType.DMA,
        ],
    )
    def k(table_hbm, src_hbm, et_hbm, dst_hbm, out_hbm,
          gidx_v, dst_v, et_stage, rows0, rows1, acc,
          sg0, sg1, ss0, ss1, so):
        c = lax.axis_index("c")
        s = lax.axis_index("s")
        wid = s * _NC + c

        def zrow(i, carry):
            for j in range(d // 16):
                rows0[i, pl.ds(j * 16, 16)] = jnp.zeros((16,), jnp.float32)
            return carry
        lax.fori_loop(0, _G, zrow, 0)
        zd = [pltpu.async_copy(rows0, acc.at[pl.ds(s * _SLAB + i * _G, _G)], so)
              for i in range(_SLAB // _G)]
        for dsc in zd:
            dsc.wait()
        plsc.subcore_barrier()

        for ph in range(2):
            base_g = wid * _GW + ph * _HGW
            pltpu.sync_copy(src_hbm.at[pl.ds(base_g, _HGW)], gidx_v)
            pltpu.sync_copy(dst_hbm.at[pl.ds(base_g, _HGW)], dst_v)

            def gchunk(ci, carry):
                pltpu.sync_copy(et_hbm.at[pl.ds(base_g + ci * 8, 8)], et_stage)

                def gidx(g, carry2):
                    for j in range(_G // 16):
                        sv = gidx_v[ci * 8 + g, pl.ds(j * 16, 16)]
                        tv = et_stage[g, pl.ds(j * 16, 16)]
                        gidx_v[ci * 8 + g, pl.ds(j * 16, 16)] = sv * _R + tv
                    return carry2
                lax.fori_loop(0, 8, gidx, 0)
                return carry
            lax.fori_loop(0, _HGW // 8, gchunk, 0)

            pltpu.async_copy(table_hbm.at[gidx_v.at[0]], rows0, sg0)
            pltpu.async_copy(table_hbm.at[gidx_v.at[1]], rows1, sg1)

            def body(g2, carry):
                g = 2 * g2
                pltpu.make_async_copy(
                    table_hbm.at[gidx_v.at[g]], rows0, sg0).wait()
                s0 = pltpu.async_copy(rows0, acc.at[dst_v.at[g]], ss0,
                                      add=True)
                pltpu.make_async_copy(
                    table_hbm.at[gidx_v.at[g + 1]], rows1, sg1).wait()
                s1 = pltpu.async_copy(rows1, acc.at[dst_v.at[g + 1]], ss1,
                                      add=True)
                s0.wait()
                pltpu.async_copy(table_hbm.at[gidx_v.at[g + 2]], rows0, sg0)
                s1.wait()
                pltpu.async_copy(table_hbm.at[gidx_v.at[g + 3]], rows1, sg1)
                return carry
            lax.fori_loop(0, _HGW // 2 - 1, body, 0)

            g = _HGW - 2
            pltpu.make_async_copy(table_hbm.at[gidx_v.at[g]], rows0, sg0).wait()
            pltpu.sync_copy(rows0, acc.at[dst_v.at[g]], add=True)
            pltpu.make_async_copy(table_hbm.at[gidx_v.at[g + 1]], rows1, sg1).wait()
            pltpu.sync_copy(rows1, acc.at[dst_v.at[g + 1]], add=True)

        plsc.subcore_barrier()

        od = [pltpu.async_copy(acc.at[pl.ds(s * _SLAB + i * _G, _G)],
                               out_hbm.at[c, pl.ds(s * _SLAB + i * _G, _G)], so)
              for i in range(_SLAB // _G)]
        for dsc in od:
            dsc.wait()

    return k


def _mm1_kernel(x_ref, w_ref, o_ref):
    o_ref[...] = jnp.dot(x_ref[...], w_ref[...],
                         preferred_element_type=jnp.float32)


def _layer2_kernel(inv_n, nclass, p_ref, b_ref, w_ref, o_ref):
    h = jnp.maximum((p_ref[0] + p_ref[1]) * inv_n + b_ref[...], 0.0)
    t = jnp.dot(h, w_ref[...], preferred_element_type=jnp.float32)
    z = jnp.zeros((h.shape[0], 128 - nclass), jnp.float32)
    for r in range(_R):
        o_ref[:, r * 128:r * 128 + nclass] = t[:, r * nclass:(r + 1) * nclass]
        o_ref[:, r * 128 + nclass:(r + 1) * 128] = z


def _final_kernel(inv_n, nclass, q_ref, b_ref, o_ref):
    o = (q_ref[0, :, :nclass] + q_ref[1, :, :nclass]) * inv_n + b_ref[...]
    m = jnp.max(o, axis=1, keepdims=True)
    lse = jnp.log(jnp.sum(jnp.exp(o - m), axis=1, keepdims=True)) + m
    o_ref[...] = o - lse


def kernel(x, edge_index, edge_type, W1, b1, W2, b2):
    nfeat = x.shape[1]
    nhid = W1.shape[2]
    nclass = W2.shape[2]
    inv_n = 1.0 / float(_NUM_NEIGHBORS)

    pad = _E_PAD - _E
    src = jnp.concatenate([edge_index[0].astype(jnp.int32),
                           jnp.zeros((pad,), jnp.int32)]).reshape(_NW * _GW, _G)
    dst = jnp.concatenate([edge_index[1].astype(jnp.int32),
                           jnp.full((pad,), _N, jnp.int32)]).reshape(_NW * _GW, _G)
    et = jnp.concatenate([edge_type.astype(jnp.int32),
                          jnp.zeros((pad,), jnp.int32)]).reshape(_NW * _GW, _G)
    W1f = W1.transpose(1, 0, 2).reshape(nfeat, _R * nhid)
    W2f = W2.transpose(1, 0, 2).reshape(nhid, _R * nclass)

    t1 = pl.pallas_call(
        _mm1_kernel,
        grid=(10,),
        in_specs=[pl.BlockSpec((_N // 10, nfeat), lambda i: (i, 0)),
                  pl.BlockSpec((nfeat, _R * nhid), lambda i: (0, 0))],
        out_specs=pl.BlockSpec((_N // 10, _R * nhid), lambda i: (i, 0)),
        out_shape=jax.ShapeDtypeStruct((_N, _R * nhid), jnp.float32),
    )(x, W1f)
    table1 = t1.reshape(_N * _R, nhid)

    p = _sc_aggregate(nhid)(table1, src, et, dst)

    t2 = pl.pallas_call(
        functools.partial(_layer2_kernel, inv_n, nclass),
        grid=(10,),
        in_specs=[pl.BlockSpec((_NC, _N_ACC // 10, nhid), lambda i: (0, i, 0)),
                  pl.BlockSpec((1, nhid), lambda i: (0, 0)),
                  pl.BlockSpec((nhid, _R * nclass), lambda i: (0, 0))],
        out_specs=pl.BlockSpec((_N_ACC // 10, _R * 128), lambda i: (i, 0)),
        out_shape=jax.ShapeDtypeStruct((_N_ACC, _R * 128), jnp.float32),
    )(p, b1.reshape(1, nhid), W2f)
    table2 = t2.reshape(_N_ACC * _R, 128)

    q = _sc_aggregate(128)(table2, src, et, dst)

    out = pl.pallas_call(
        functools.partial(_final_kernel, inv_n, nclass),
        grid=(10,),
        in_specs=[pl.BlockSpec((_NC, _N_ACC // 10, 128), lambda i: (0, i, 0)),
                  pl.BlockSpec((1, nclass), lambda i: (0, 0))],
        out_specs=pl.BlockSpec((_N_ACC // 10, nclass), lambda i: (i, 0)),
        out_shape=jax.ShapeDtypeStruct((_N_ACC, nclass), jnp.float32),
    )(q, b2.reshape(1, nclass))
    return out[:_N]


# speedup vs baseline: 13.3930x; 1.1837x over previous
import functools

import jax
import jax.numpy as jnp
from jax import lax
from jax.experimental import pallas as pl
from jax.experimental.pallas import tpu as pltpu
from jax.experimental.pallas import tpu_sc as plsc

_N = 10000
_E = 320000
_R = 4
_NUM_NEIGHBORS = 32

_NC = 2
_NS = 16
_NW = _NC * _NS

_G = 128
_E_PAD = 327680
_GS = 160
_GW0 = 120
_GW1 = _GS - _GW0
_PH = 5
_N_ACC = 10240
_SLAB = _N_ACC // _NS


def _sc_aggregate(d: int):
    mesh = plsc.VectorSubcoreMesh(core_axis_name="c", subcore_axis_name="s")

    ppg = _GW0 // _PH
    ppg1 = _GW1 // _PH

    @functools.partial(
        pl.kernel,
        out_type=jax.ShapeDtypeStruct((_NC, _N_ACC, d), jnp.float32),
        mesh=mesh,
        scratch_types=[
            pltpu.VMEM((ppg, _G), jnp.int32),
            pltpu.VMEM((ppg, _G), jnp.int32),
            pltpu.VMEM((ppg, _G), jnp.int32),
            pltpu.VMEM((_G, d), jnp.float32),
            pltpu.VMEM((_G, d), jnp.float32),
            pltpu.VMEM_SHARED((_N_ACC, d), jnp.float32),
            pltpu.SemaphoreType.DMA,
            pltpu.SemaphoreType.DMA,
            pltpu.SemaphoreType.DMA,
            pltpu.SemaphoreType.DMA,
            pltpu.SemaphoreType.DMA,
        ],
    )
    def k(table_hbm, src_hbm, et_hbm, dst_hbm, out_hbm,
          gidx_v, et_v, dst_v, rows0, rows1, acc,
          sg0, sg1, ss0, ss1, so):
        c = lax.axis_index("c")
        s = lax.axis_index("s")

        def zrow(i, carry):
            for j in range(d // 16):
                rows0[i, pl.ds(j * 16, 16)] = jnp.zeros((16,), jnp.float32)
            return carry
        lax.fori_loop(0, _G, zrow, 0)
        zd = [pltpu.async_copy(rows0, acc.at[pl.ds(s * _SLAB + i * _G, _G)], so)
              for i in range(_SLAB // _G)]
        for dsc in zd:
            dsc.wait()
        plsc.subcore_barrier()

        def process(base_g, ngroups):
            pltpu.sync_copy(src_hbm.at[pl.ds(base_g, ngroups)],
                            gidx_v.at[pl.ds(0, ngroups)])
            pltpu.sync_copy(et_hbm.at[pl.ds(base_g, ngroups)],
                            et_v.at[pl.ds(0, ngroups)])
            pltpu.sync_copy(dst_hbm.at[pl.ds(base_g, ngroups)],
                            dst_v.at[pl.ds(0, ngroups)])

            def gi(g, carry):
                for j in range(_G // 16):
                    sv = gidx_v[g, pl.ds(j * 16, 16)]
                    tv = et_v[g, pl.ds(j * 16, 16)]
                    gidx_v[g, pl.ds(j * 16, 16)] = sv * _R + tv
                return carry
            lax.fori_loop(0, ngroups, gi, 0)

            pltpu.async_copy(table_hbm.at[gidx_v.at[0]], rows0, sg0)
            pltpu.async_copy(table_hbm.at[gidx_v.at[1]], rows1, sg1)

            def body(g2, carry):
                g = 2 * g2
                pltpu.make_async_copy(
                    table_hbm.at[gidx_v.at[g]], rows0, sg0).wait()
                s0 = pltpu.async_copy(rows0, acc.at[dst_v.at[g]], ss0,
                                      add=True)
                pltpu.make_async_copy(
                    table_hbm.at[gidx_v.at[g + 1]], rows1, sg1).wait()
                s1 = pltpu.async_copy(rows1, acc.at[dst_v.at[g + 1]], ss1,
                                      add=True)
                s0.wait()
                pltpu.async_copy(table_hbm.at[gidx_v.at[g + 2]], rows0, sg0)
                s1.wait()
                pltpu.async_copy(table_hbm.at[gidx_v.at[g + 3]], rows1, sg1)
                return carry
            lax.fori_loop(0, ngroups // 2 - 1, body, 0)

            g = ngroups - 2
            pltpu.make_async_copy(table_hbm.at[gidx_v.at[g]], rows0, sg0).wait()
            pltpu.sync_copy(rows0, acc.at[dst_v.at[g]], add=True)
            pltpu.make_async_copy(table_hbm.at[gidx_v.at[g + 1]], rows1, sg1).wait()
            pltpu.sync_copy(rows1, acc.at[dst_v.at[g + 1]], add=True)

        @pl.when(c == 0)
        def _():
            for ph in range(_PH):
                process(s * _GS + ph * ppg, ppg)

        @pl.when(c == 1)
        def _():
            for ph in range(_PH):
                process(s * _GS + _GW0 + ph * ppg1, ppg1)

        plsc.subcore_barrier()

        od = [pltpu.async_copy(acc.at[pl.ds(s * _SLAB + i * _G, _G)],
                               out_hbm.at[c, pl.ds(s * _SLAB + i * _G, _G)], so)
              for i in range(_SLAB // _G)]
        for dsc in od:
            dsc.wait()

    return k


def _mm1_kernel(x_ref, w_ref, o_ref):
    o_ref[...] = jnp.dot(x_ref[...], w_ref[...],
                         preferred_element_type=jnp.float32)


def _layer2_kernel(inv_n, nclass, p_ref, b_ref, w_ref, o_ref):
    h = jnp.maximum((p_ref[0] + p_ref[1]) * inv_n + b_ref[...], 0.0)
    t = jnp.dot(h, w_ref[...], preferred_element_type=jnp.float32)
    z = jnp.zeros((h.shape[0], 128 - nclass), jnp.float32)
    for r in range(_R):
        o_ref[:, r * 128:r * 128 + nclass] = t[:, r * nclass:(r + 1) * nclass]
        o_ref[:, r * 128 + nclass:(r + 1) * 128] = z


def _final_kernel(inv_n, nclass, q_ref, b_ref, o_ref):
    o = (q_ref[0, :, :nclass] + q_ref[1, :, :nclass]) * inv_n + b_ref[...]
    m = jnp.max(o, axis=1, keepdims=True)
    lse = jnp.log(jnp.sum(jnp.exp(o - m), axis=1, keepdims=True)) + m
    o_ref[...] = o - lse


def kernel(x, edge_index, edge_type, W1, b1, W2, b2):
    nfeat = x.shape[1]
    nhid = W1.shape[2]
    nclass = W2.shape[2]
    inv_n = 1.0 / float(_NUM_NEIGHBORS)

    pad = _E_PAD - _E
    src = jnp.concatenate([edge_index[0].astype(jnp.int32),
                           jnp.zeros((pad,), jnp.int32)]).reshape(_E_PAD // _G, _G)
    dst = jnp.concatenate([edge_index[1].astype(jnp.int32),
                           jnp.full((pad,), _N, jnp.int32)]).reshape(_E_PAD // _G, _G)
    et = jnp.concatenate([edge_type.astype(jnp.int32),
                          jnp.zeros((pad,), jnp.int32)]).reshape(_E_PAD // _G, _G)
    W1f = W1.transpose(1, 0, 2).reshape(nfeat, _R * nhid)
    W2f = W2.transpose(1, 0, 2).reshape(nhid, _R * nclass)

    t1 = pl.pallas_call(
        _mm1_kernel,
        grid=(10,),
        in_specs=[pl.BlockSpec((_N // 10, nfeat), lambda i: (i, 0)),
                  pl.BlockSpec((nfeat, _R * nhid), lambda i: (0, 0))],
        out_specs=pl.BlockSpec((_N // 10, _R * nhid), lambda i: (i, 0)),
        out_shape=jax.ShapeDtypeStruct((_N, _R * nhid), jnp.float32),
    )(x, W1f)
    table1 = t1.reshape(_N * _R, nhid)

    p = _sc_aggregate(nhid)(table1, src, et, dst)

    t2 = pl.pallas_call(
        functools.partial(_layer2_kernel, inv_n, nclass),
        grid=(10,),
        in_specs=[pl.BlockSpec((_NC, _N_ACC // 10, nhid), lambda i: (0, i, 0)),
                  pl.BlockSpec((1, nhid), lambda i: (0, 0)),
                  pl.BlockSpec((nhid, _R * nclass), lambda i: (0, 0))],
        out_specs=pl.BlockSpec((_N_ACC // 10, _R * 128), lambda i: (i, 0)),
        out_shape=jax.ShapeDtypeStruct((_N_ACC, _R * 128), jnp.float32),
    )(p, b1.reshape(1, nhid), W2f)
    table2 = t2.reshape(_N_ACC * _R, 128)

    q = _sc_aggregate(128)(table2, src, et, dst)

    out = pl.pallas_call(
        functools.partial(_final_kernel, inv_n, nclass),
        grid=(10,),
        in_specs=[pl.BlockSpec((_NC, _N_ACC // 10, 128), lambda i: (0, i, 0)),
                  pl.BlockSpec((1, nclass), lambda i: (0, 0))],
        out_specs=pl.BlockSpec((_N_ACC // 10, nclass), lambda i: (i, 0)),
        out_shape=jax.ShapeDtypeStruct((_N_ACC, nclass), jnp.float32),
    )(q, b2.reshape(1, nclass))
    return out[:_N]
